# baseline (speedup 1.0000x reference)
import jax
import jax.numpy as jnp
from jax.experimental import pallas as pl


def _bias_add_kernel(x_ref, b_ref, o_ref):
    o_ref[...] = x_ref[...] + b_ref[...]


def kernel(user_emb, loc_logits, user_loc_weights, loc_bias):
    B, L = loc_logits.shape
    R = 8
    bias2d = loc_bias.reshape(1, L)
    out = pl.pallas_call(
        _bias_add_kernel,
        grid=(B // R,),
        in_specs=[
            pl.BlockSpec((R, L), lambda i: (i, 0)),
            pl.BlockSpec((1, L), lambda i: (0, 0)),
        ],
        out_specs=pl.BlockSpec((R, L), lambda i: (i, 0)),
        out_shape=jax.ShapeDtypeStruct((B, L), jnp.float32),
    )(loc_logits, bias2d)
    return out


# speedup vs baseline: 1.0059x; 1.0059x over previous
import jax
import jax.numpy as jnp
from jax.experimental import pallas as pl


def _bias_add_kernel(x_ref, b_ref, o_ref):
    o_ref[...] = x_ref[...] + b_ref[...]


def kernel(user_emb, loc_logits, user_loc_weights, loc_bias):
    B, L = loc_logits.shape
    R = 32
    bias2d = loc_bias.reshape(1, L)
    out = pl.pallas_call(
        _bias_add_kernel,
        grid=(B // R,),
        in_specs=[
            pl.BlockSpec((R, L), lambda i: (i, 0)),
            pl.BlockSpec((1, L), lambda i: (0, 0)),
        ],
        out_specs=pl.BlockSpec((R, L), lambda i: (i, 0)),
        out_shape=jax.ShapeDtypeStruct((B, L), jnp.float32),
    )(loc_logits, bias2d)
    return out
